# Initial kernel scaffold; baseline (speedup 1.0000x reference)
import jax
import jax.numpy as jnp
from jax.experimental import pallas as pl


def kernel(feat, edge_index, W1, W2, We1, be1, We2, be2):
    raise NotImplementedError("write your pallas kernel here")


# speedup vs baseline: 2.8257x; 2.8257x over previous
import functools

import jax
import jax.numpy as jnp
from jax import lax
from jax.experimental import pallas as pl
from jax.experimental.pallas import tpu as pltpu
from jax.experimental.pallas import tpu_sc as plsc

_N = 10000
_E = 160000
_HALF = _E // 2
_D = 256
_DQ = 64
_H = 64
_C = 10
_NSUB = 16
_NCORE = 2
_LANES = 16

_EPT = _E // _NSUB
_CH = 400
_NCHUNK = _EPT // _CH
_RPB = 624
_RTAIL = _N - _NSUB * _RPB

_EPW = _E // (_NSUB * _NCORE)
_CH2 = 1000
_NCHUNK2 = _EPW // _CH2

_BLKN = 1000
_BLKE = 8000


def _make_segsum(masked: bool):
    mesh = plsc.VectorSubcoreMesh(core_axis_name="c", subcore_axis_name="s")
    out_type = [jax.ShapeDtypeStruct((_N, _DQ), jnp.float32)
                for _ in range(4)]
    scratch = [pltpu.VMEM((_CH,), jnp.int32),
               pltpu.VMEM((_CH,), jnp.int32),
               pltpu.VMEM((_CH,), jnp.float32),
               pltpu.VMEM((_CH, _DQ), jnp.float32),
               pltpu.VMEM_SHARED((_N, _DQ), jnp.float32),
               pltpu.SemaphoreType.DMA]

    @functools.partial(
        pl.kernel, mesh=mesh, out_type=out_type, scratch_types=scratch,
        compiler_params=pltpu.CompilerParams(use_tc_tiling_on_sc=False,
                                             needs_layout_passes=False))
    def body(f0, f1, f2, f3, src, dst, mh, zero, o0, o1, o2, o3,
             sidx, didx, mrow, rows, acc, sem):
        c = lax.axis_index("c")
        s = lax.axis_index("s")
        r0 = pl.multiple_of(s * _RPB, 8)

        def run(tbl, out):
            pltpu.sync_copy(zero.at[pl.ds(r0, _RPB)], acc.at[pl.ds(r0, _RPB)])

            @pl.when(s == _NSUB - 1)
            def _():
                pltpu.sync_copy(zero.at[pl.ds(_NSUB * _RPB, _RTAIL)],
                                acc.at[pl.ds(_NSUB * _RPB, _RTAIL)])

            plsc.subcore_barrier()
            for i in range(_NCHUNK):
                base = pl.multiple_of(s * _EPT + i * _CH, 8)
                pltpu.sync_copy(src.at[pl.ds(base, _CH)], sidx)
                pltpu.sync_copy(dst.at[pl.ds(base, _CH)], didx)
                pltpu.async_copy(tbl.at[sidx], rows, sem).wait()
                if masked:
                    mbase = pl.multiple_of(
                        jnp.where(base >= _HALF, base - _HALF, base), 8)
                    pltpu.sync_copy(mh.at[pl.ds(mbase, _CH)], mrow)

                    def scale(e, carry):
                        mvec = plsc.load_gather(
                            mrow, [jnp.full((_LANES,), e, jnp.int32)])
                        for j in range(_DQ // _LANES):
                            sl = pl.ds(j * _LANES, _LANES)
                            rows[e, sl] = rows[e, sl] * mvec
                        return carry

                    lax.fori_loop(0, _CH, scale, 0)
                pltpu.sync_copy(rows, acc.at[didx], add=True)
            plsc.subcore_barrier()
            pltpu.sync_copy(acc.at[pl.ds(r0, _RPB)], out.at[pl.ds(r0, _RPB)])

            @pl.when(s == _NSUB - 1)
            def _():
                pltpu.sync_copy(acc.at[pl.ds(_NSUB * _RPB, _RTAIL)],
                                out.at[pl.ds(_NSUB * _RPB, _RTAIL)])

            plsc.subcore_barrier()

        @pl.when(c == 0)
        def _():
            run(f0, o0)
            run(f1, o1)

        @pl.when(c == 1)
        def _():
            run(f2, o2)
            run(f3, o3)

    return body


_segsum_plain = _make_segsum(False)
_segsum_masked = _make_segsum(True)


def _edge_gather(A, B, src, dst):
    mesh = plsc.VectorSubcoreMesh(core_axis_name="c", subcore_axis_name="s")
    out_type = [jax.ShapeDtypeStruct((_E, _H), jnp.float32),
                jax.ShapeDtypeStruct((_E, _H), jnp.float32)]
    scratch = [pltpu.VMEM((_CH2,), jnp.int32),
               pltpu.VMEM((_CH2, _H), jnp.float32),
               pltpu.SemaphoreType.DMA]

    @functools.partial(
        pl.kernel, mesh=mesh, out_type=out_type, scratch_types=scratch,
        compiler_params=pltpu.CompilerParams(use_tc_tiling_on_sc=False,
                                             needs_layout_passes=False))
    def body(A_h, B_h, src_h, dst_h, As_o, Bd_o, idx, rows, sem):
        c = lax.axis_index("c")
        s = lax.axis_index("s")
        w = s * _NCORE + c
        for i in range(_NCHUNK2):
            base = pl.multiple_of(w * _EPW + i * _CH2, 8)
            pltpu.sync_copy(src_h.at[pl.ds(base, _CH2)], idx)
            pltpu.async_copy(A_h.at[idx], rows, sem).wait()
            pltpu.sync_copy(rows, As_o.at[pl.ds(base, _CH2)])
            pltpu.sync_copy(dst_h.at[pl.ds(base, _CH2)], idx)
            pltpu.async_copy(B_h.at[idx], rows, sem).wait()
            pltpu.sync_copy(rows, Bd_o.at[pl.ds(base, _CH2)])

    return body(A, B, src, dst)


def _node_mlp_body(a0, a1, a2, a3, W1, We1, A_o, B_o):
    w1 = W1[...]
    we1 = We1[...]
    e = jnp.maximum(
        jnp.dot(a0[...], w1[0 * _DQ:1 * _DQ], preferred_element_type=jnp.float32)
        + jnp.dot(a1[...], w1[1 * _DQ:2 * _DQ], preferred_element_type=jnp.float32)
        + jnp.dot(a2[...], w1[2 * _DQ:3 * _DQ], preferred_element_type=jnp.float32)
        + jnp.dot(a3[...], w1[3 * _DQ:4 * _DQ], preferred_element_type=jnp.float32),
        0.0)
    A_o[...] = jnp.dot(e, we1[:_D], preferred_element_type=jnp.float32)
    B_o[...] = jnp.dot(e, we1[_D:], preferred_element_type=jnp.float32)


def _node_mlp(aggs, W1, We1):
    n_blk = _N // _BLKN
    qspec = pl.BlockSpec((_BLKN, _DQ), lambda i: (i, 0))
    return pl.pallas_call(
        _node_mlp_body,
        grid=(n_blk,),
        in_specs=[
            qspec, qspec, qspec, qspec,
            pl.BlockSpec((_D, _D), lambda i: (0, 0)),
            pl.BlockSpec((2 * _D, _H), lambda i: (0, 0)),
        ],
        out_specs=[
            pl.BlockSpec((_BLKN, _H), lambda i: (i, 0)),
            pl.BlockSpec((_BLKN, _H), lambda i: (i, 0)),
        ],
        out_shape=[jax.ShapeDtypeStruct((_N, _H), jnp.float32),
                   jax.ShapeDtypeStruct((_N, _H), jnp.float32)],
    )(*aggs, W1, We1)


def _mask_mlp_body(As1, Bd1, As2, Bd2, be1, we2, be2, out):
    h1 = jnp.maximum(As1[...] + Bd1[...] + be1[...], 0.0)
    v1 = jnp.sum(h1 * we2[...], axis=1, keepdims=True) + be2[0, 0]
    h2 = jnp.maximum(As2[...] + Bd2[...] + be1[...], 0.0)
    v2 = jnp.sum(h2 * we2[...], axis=1, keepdims=True) + be2[0, 0]
    out[...] = (jax.nn.sigmoid(v1) + jax.nn.sigmoid(v2)) * 0.5


def _mask_mlp(As, Bd, be1, We2, be2):
    n_blk = _HALF // _BLKE
    return pl.pallas_call(
        _mask_mlp_body,
        grid=(n_blk,),
        in_specs=[
            pl.BlockSpec((_BLKE, _H), lambda i: (i, 0)),
            pl.BlockSpec((_BLKE, _H), lambda i: (i, 0)),
            pl.BlockSpec((_BLKE, _H), lambda i: (i + n_blk, 0)),
            pl.BlockSpec((_BLKE, _H), lambda i: (i + n_blk, 0)),
            pl.BlockSpec((1, _H), lambda i: (0, 0)),
            pl.BlockSpec((1, _H), lambda i: (0, 0)),
            pl.BlockSpec((1, 1), lambda i: (0, 0)),
        ],
        out_specs=pl.BlockSpec((_BLKE, 1), lambda i: (i, 0)),
        out_shape=jax.ShapeDtypeStruct((_HALF, 1), jnp.float32),
    )(As, Bd, As, Bd, be1, We2, be2)


def _final_body(a0, a1, a2, a3, W1, W2, out, acc):
    i = pl.program_id(0)
    w1 = W1[...]
    h = jnp.maximum(
        jnp.dot(a0[...], w1[0 * _DQ:1 * _DQ], preferred_element_type=jnp.float32)
        + jnp.dot(a1[...], w1[1 * _DQ:2 * _DQ], preferred_element_type=jnp.float32)
        + jnp.dot(a2[...], w1[2 * _DQ:3 * _DQ], preferred_element_type=jnp.float32)
        + jnp.dot(a3[...], w1[3 * _DQ:4 * _DQ], preferred_element_type=jnp.float32),
        0.0)
    cs = jnp.sum(h, axis=0, keepdims=True)

    @pl.when(i == 0)
    def _():
        acc[...] = cs

    @pl.when(i > 0)
    def _():
        acc[...] = acc[...] + cs

    @pl.when(i == pl.num_programs(0) - 1)
    def _():
        pooled = acc[...] * jnp.float32(1.0 / _N)
        logits = jnp.dot(pooled, W2[...], preferred_element_type=jnp.float32)
        m = jnp.max(logits)
        ex = jnp.exp(logits - m)
        out[...] = ex / jnp.sum(ex)


def _final(aggs, W1, W2):
    n_blk = _N // _BLKN
    qspec = pl.BlockSpec((_BLKN, _DQ), lambda i: (i, 0))
    return pl.pallas_call(
        _final_body,
        grid=(n_blk,),
        in_specs=[
            qspec, qspec, qspec, qspec,
            pl.BlockSpec((_D, _D), lambda i: (0, 0)),
            pl.BlockSpec((_D, _C), lambda i: (0, 0)),
        ],
        out_specs=pl.BlockSpec((1, _C), lambda i: (0, 0)),
        out_shape=jax.ShapeDtypeStruct((1, _C), jnp.float32),
        scratch_shapes=[pltpu.VMEM((1, _D), jnp.float32)],
    )(*aggs, W1, W2)


def kernel(feat, edge_index, W1, W2, We1, be1, We2, be2):
    src = edge_index[0].astype(jnp.int32)
    dst = edge_index[1].astype(jnp.int32)
    fq = [feat[:, k * _DQ:(k + 1) * _DQ] for k in range(4)]
    zero = jnp.zeros((_N, _DQ), jnp.float32)
    dummy_m = jnp.zeros((_HALF,), jnp.float32)

    aggs = _segsum_plain(*fq, src, dst, dummy_m, zero)
    A, B = _node_mlp(aggs, W1, We1)
    As, Bd = _edge_gather(A, B, src, dst)
    m2 = _mask_mlp(As, Bd, be1.reshape(1, _H), We2.reshape(1, _H),
                   be2.reshape(1, 1))
    m_h = m2.reshape(-1)
    edge_mask = jnp.concatenate([m_h, m_h])
    a2 = _segsum_masked(*fq, src, dst, m_h, zero)
    probs = _final(a2, W1, W2).reshape(-1)
    return probs, edge_mask


# speedup vs baseline: 4.0025x; 1.4164x over previous
import functools

import jax
import jax.numpy as jnp
from jax import lax
from jax.experimental import pallas as pl
from jax.experimental.pallas import tpu as pltpu
from jax.experimental.pallas import tpu_sc as plsc

_N = 10000
_E = 160000
_HALF = _E // 2
_D = 256
_DQ = 64
_H = 64
_C = 10
_NSUB = 16
_NCORE = 2
_LANES = 16

_EPT = _E // _NSUB
_CH = 400
_NCHUNK = _EPT // _CH
_RPB = 624
_RTAIL = _N - _NSUB * _RPB

_CH2 = 200
_NCHUNK2 = _HALF // (_CH2 * _NSUB)

_BLKN = 1000
_BLKE = 8000


def _make_segsum(masked: bool):
    mesh = plsc.VectorSubcoreMesh(core_axis_name="c", subcore_axis_name="s")
    out_type = [jax.ShapeDtypeStruct((_N, _DQ), jnp.float32)
                for _ in range(4)]
    scratch = [pltpu.VMEM((_CH,), jnp.int32),
               pltpu.VMEM((_CH,), jnp.int32),
               pltpu.VMEM((_CH,), jnp.int32),
               pltpu.VMEM((_CH,), jnp.float32),
               pltpu.VMEM((_CH, _DQ), jnp.float32),
               pltpu.VMEM((_CH, _DQ), jnp.float32),
               pltpu.VMEM_SHARED((_N, _DQ), jnp.float32),
               pltpu.SemaphoreType.DMA,
               pltpu.SemaphoreType.DMA]

    @functools.partial(
        pl.kernel, mesh=mesh, out_type=out_type, scratch_types=scratch,
        compiler_params=pltpu.CompilerParams(use_tc_tiling_on_sc=False,
                                             needs_layout_passes=False))
    def body(f0, f1, f2, f3, src, dst, mh, zero, o0, o1, o2, o3,
             sidx0, sidx1, didx, mrow, rows0, rows1, acc, sem0, sem1):
        c = lax.axis_index("c")
        s = lax.axis_index("s")
        r0 = pl.multiple_of(s * _RPB, 8)
        sidx = [sidx0, sidx1]
        rows = [rows0, rows1]
        sems = [sem0, sem1]

        def run(tbl, out):
            pltpu.sync_copy(zero.at[pl.ds(r0, _RPB)], acc.at[pl.ds(r0, _RPB)])

            @pl.when(s == _NSUB - 1)
            def _():
                pltpu.sync_copy(zero.at[pl.ds(_NSUB * _RPB, _RTAIL)],
                                acc.at[pl.ds(_NSUB * _RPB, _RTAIL)])

            plsc.subcore_barrier()
            base0 = pl.multiple_of(s * _EPT, 8)
            pltpu.sync_copy(src.at[pl.ds(base0, _CH)], sidx[0])
            handles = [pltpu.async_copy(tbl.at[sidx[0]], rows[0], sems[0]),
                       None]
            for i in range(_NCHUNK):
                cur = i % 2
                nxt = 1 - cur
                if i + 1 < _NCHUNK:
                    basen = pl.multiple_of(s * _EPT + (i + 1) * _CH, 8)
                    pltpu.sync_copy(src.at[pl.ds(basen, _CH)], sidx[nxt])
                    handles[nxt] = pltpu.async_copy(tbl.at[sidx[nxt]],
                                                    rows[nxt], sems[nxt])
                base = pl.multiple_of(s * _EPT + i * _CH, 8)
                pltpu.sync_copy(dst.at[pl.ds(base, _CH)], didx)
                handles[cur].wait()
                if masked:
                    mbase = pl.multiple_of(
                        jnp.where(base >= _HALF, base - _HALF, base), 8)
                    pltpu.sync_copy(mh.at[pl.ds(mbase, _CH)], mrow)
                    rc = rows[cur]

                    def scale(e, carry):
                        mvec = plsc.load_gather(
                            mrow, [jnp.full((_LANES,), e, jnp.int32)])
                        for j in range(_DQ // _LANES):
                            sl = pl.ds(j * _LANES, _LANES)
                            rc[e, sl] = rc[e, sl] * mvec
                        return carry

                    lax.fori_loop(0, _CH, scale, 0)
                pltpu.sync_copy(rows[cur], acc.at[didx], add=True)
            plsc.subcore_barrier()
            pltpu.sync_copy(acc.at[pl.ds(r0, _RPB)], out.at[pl.ds(r0, _RPB)])

            @pl.when(s == _NSUB - 1)
            def _():
                pltpu.sync_copy(acc.at[pl.ds(_NSUB * _RPB, _RTAIL)],
                                out.at[pl.ds(_NSUB * _RPB, _RTAIL)])

            plsc.subcore_barrier()

        @pl.when(c == 0)
        def _():
            run(f0, o0)
            run(f1, o1)

        @pl.when(c == 1)
        def _():
            run(f2, o2)
            run(f3, o3)

    return body


_segsum_plain = _make_segsum(False)
_segsum_masked = _make_segsum(True)


def _edge_gather(AB, src, dst):
    mesh = plsc.VectorSubcoreMesh(core_axis_name="c", subcore_axis_name="s")
    out_type = [jax.ShapeDtypeStruct((_HALF, 2 * _H), jnp.float32),
                jax.ShapeDtypeStruct((_HALF, 2 * _H), jnp.float32)]
    scratch = [pltpu.VMEM((_CH2,), jnp.int32),
               pltpu.VMEM((_CH2,), jnp.int32),
               pltpu.VMEM((_CH2, 2 * _H), jnp.float32),
               pltpu.VMEM((_CH2, 2 * _H), jnp.float32),
               pltpu.SemaphoreType.DMA,
               pltpu.SemaphoreType.DMA]

    @functools.partial(
        pl.kernel, mesh=mesh, out_type=out_type, scratch_types=scratch,
        compiler_params=pltpu.CompilerParams(use_tc_tiling_on_sc=False,
                                             needs_layout_passes=False))
    def body(AB_h, src_h, dst_h, ABs_o, ABd_o,
             idx0, idx1, rows0, rows1, sem0, sem1):
        c = lax.axis_index("c")
        s = lax.axis_index("s")
        u = s
        idxs = [idx0, idx1]
        rows = [rows0, rows1]
        sems = [sem0, sem1]

        def run(eidx, out):
            base0 = pl.multiple_of(u * _CH2, 8)
            pltpu.sync_copy(eidx.at[pl.ds(base0, _CH2)], idxs[0])
            handles = [pltpu.async_copy(AB_h.at[idxs[0]], rows[0], sems[0]),
                       None]
            for k in range(_NCHUNK2):
                cur = k % 2
                nxt = 1 - cur
                if k + 1 < _NCHUNK2:
                    basen = pl.multiple_of((u + (k + 1) * _NSUB) * _CH2, 8)
                    pltpu.sync_copy(eidx.at[pl.ds(basen, _CH2)], idxs[nxt])
                    handles[nxt] = pltpu.async_copy(AB_h.at[idxs[nxt]],
                                                    rows[nxt], sems[nxt])
                base = pl.multiple_of((u + k * _NSUB) * _CH2, 8)
                handles[cur].wait()
                pltpu.sync_copy(rows[cur], out.at[pl.ds(base, _CH2)])

        @pl.when(c == 0)
        def _():
            run(src_h, ABs_o)

        @pl.when(c == 1)
        def _():
            run(dst_h, ABd_o)

    return body(AB, src, dst)


def _node_mlp_body(a0, a1, a2, a3, W1, We1, AB_o):
    w1 = W1[...]
    we1 = We1[...]
    e = jnp.maximum(
        jnp.dot(a0[...], w1[0 * _DQ:1 * _DQ], preferred_element_type=jnp.float32)
        + jnp.dot(a1[...], w1[1 * _DQ:2 * _DQ], preferred_element_type=jnp.float32)
        + jnp.dot(a2[...], w1[2 * _DQ:3 * _DQ], preferred_element_type=jnp.float32)
        + jnp.dot(a3[...], w1[3 * _DQ:4 * _DQ], preferred_element_type=jnp.float32),
        0.0)
    AB_o[:, :_H] = jnp.dot(e, we1[:_D], preferred_element_type=jnp.float32)
    AB_o[:, _H:] = jnp.dot(e, we1[_D:], preferred_element_type=jnp.float32)


def _node_mlp(aggs, W1, We1):
    n_blk = _N // _BLKN
    qspec = pl.BlockSpec((_BLKN, _DQ), lambda i: (i, 0))
    return pl.pallas_call(
        _node_mlp_body,
        grid=(n_blk,),
        in_specs=[
            qspec, qspec, qspec, qspec,
            pl.BlockSpec((_D, _D), lambda i: (0, 0)),
            pl.BlockSpec((2 * _D, _H), lambda i: (0, 0)),
        ],
        out_specs=pl.BlockSpec((_BLKN, 2 * _H), lambda i: (i, 0)),
        out_shape=jax.ShapeDtypeStruct((_N, 2 * _H), jnp.float32),
    )(*aggs, W1, We1)


def _mask_mlp_body(abs_, abd_, be1, we2, be2, out):
    h1 = jnp.maximum(abs_[:, :_H] + abd_[:, _H:] + be1[...], 0.0)
    v1 = jnp.sum(h1 * we2[...], axis=1, keepdims=True) + be2[0, 0]
    h2 = jnp.maximum(abd_[:, :_H] + abs_[:, _H:] + be1[...], 0.0)
    v2 = jnp.sum(h2 * we2[...], axis=1, keepdims=True) + be2[0, 0]
    out[...] = (jax.nn.sigmoid(v1) + jax.nn.sigmoid(v2)) * 0.5


def _mask_mlp(ABs, ABd, be1, We2, be2):
    n_blk = _HALF // _BLKE
    espec = pl.BlockSpec((_BLKE, 2 * _H), lambda i: (i, 0))
    return pl.pallas_call(
        _mask_mlp_body,
        grid=(n_blk,),
        in_specs=[
            espec, espec,
            pl.BlockSpec((1, _H), lambda i: (0, 0)),
            pl.BlockSpec((1, _H), lambda i: (0, 0)),
            pl.BlockSpec((1, 1), lambda i: (0, 0)),
        ],
        out_specs=pl.BlockSpec((_BLKE, 1), lambda i: (i, 0)),
        out_shape=jax.ShapeDtypeStruct((_HALF, 1), jnp.float32),
    )(ABs, ABd, be1, We2, be2)


def _final_body(a0, a1, a2, a3, W1, W2, out, acc):
    i = pl.program_id(0)
    w1 = W1[...]
    h = jnp.maximum(
        jnp.dot(a0[...], w1[0 * _DQ:1 * _DQ], preferred_element_type=jnp.float32)
        + jnp.dot(a1[...], w1[1 * _DQ:2 * _DQ], preferred_element_type=jnp.float32)
        + jnp.dot(a2[...], w1[2 * _DQ:3 * _DQ], preferred_element_type=jnp.float32)
        + jnp.dot(a3[...], w1[3 * _DQ:4 * _DQ], preferred_element_type=jnp.float32),
        0.0)
    cs = jnp.sum(h, axis=0, keepdims=True)

    @pl.when(i == 0)
    def _():
        acc[...] = cs

    @pl.when(i > 0)
    def _():
        acc[...] = acc[...] + cs

    @pl.when(i == pl.num_programs(0) - 1)
    def _():
        pooled = acc[...] * jnp.float32(1.0 / _N)
        logits = jnp.dot(pooled, W2[...], preferred_element_type=jnp.float32)
        m = jnp.max(logits)
        ex = jnp.exp(logits - m)
        out[...] = ex / jnp.sum(ex)


def _final(aggs, W1, W2):
    n_blk = _N // _BLKN
    qspec = pl.BlockSpec((_BLKN, _DQ), lambda i: (i, 0))
    return pl.pallas_call(
        _final_body,
        grid=(n_blk,),
        in_specs=[
            qspec, qspec, qspec, qspec,
            pl.BlockSpec((_D, _D), lambda i: (0, 0)),
            pl.BlockSpec((_D, _C), lambda i: (0, 0)),
        ],
        out_specs=pl.BlockSpec((1, _C), lambda i: (0, 0)),
        out_shape=jax.ShapeDtypeStruct((1, _C), jnp.float32),
        scratch_shapes=[pltpu.VMEM((1, _D), jnp.float32)],
    )(*aggs, W1, W2)


def kernel(feat, edge_index, W1, W2, We1, be1, We2, be2):
    src = edge_index[0].astype(jnp.int32)
    dst = edge_index[1].astype(jnp.int32)
    fq = [feat[:, k * _DQ:(k + 1) * _DQ] for k in range(4)]
    zero = jnp.zeros((_N, _DQ), jnp.float32)
    dummy_m = jnp.zeros((_HALF,), jnp.float32)

    aggs = _segsum_plain(*fq, src, dst, dummy_m, zero)
    AB = _node_mlp(aggs, W1, We1)
    ABs, ABd = _edge_gather(AB, src, dst)
    m2 = _mask_mlp(ABs, ABd, be1.reshape(1, _H), We2.reshape(1, _H),
                   be2.reshape(1, 1))
    m_h = m2.reshape(-1)
    edge_mask = jnp.concatenate([m_h, m_h])
    a2 = _segsum_masked(*fq, src, dst, m_h, zero)
    probs = _final(a2, W1, W2).reshape(-1)
    return probs, edge_mask


# speedup vs baseline: 4.2710x; 1.0671x over previous
import functools

import jax
import jax.numpy as jnp
from jax import lax
from jax.experimental import pallas as pl
from jax.experimental.pallas import tpu as pltpu
from jax.experimental.pallas import tpu_sc as plsc

_N = 10000
_E = 160000
_HALF = _E // 2
_D = 256
_DQ = 64
_H = 64
_C = 10
_NSUB = 16
_NCORE = 2
_LANES = 16

_EPT = _E // _NSUB
_CH = 400
_NCHUNK = _EPT // _CH
_RPB = 624
_RTAIL = _N - _NSUB * _RPB

_CH2 = 200
_NCHUNK2 = _HALF // (_CH2 * _NSUB)

_BLKN = 1000
_BLKE = 8000


def _make_segsum(masked: bool):
    mesh = plsc.VectorSubcoreMesh(core_axis_name="c", subcore_axis_name="s")
    out_type = [jax.ShapeDtypeStruct((_N, _DQ), jnp.float32)
                for _ in range(4)]
    scratch = [pltpu.VMEM((_CH,), jnp.int32),
               pltpu.VMEM((_CH,), jnp.int32),
               pltpu.VMEM((_CH,), jnp.int32),
               pltpu.VMEM((_CH,), jnp.int32),
               pltpu.VMEM((_CH,), jnp.float32),
               pltpu.VMEM((_CH, _DQ), jnp.float32),
               pltpu.VMEM((_CH, _DQ), jnp.float32),
               pltpu.VMEM_SHARED((_N, _DQ), jnp.float32),
               pltpu.SemaphoreType.DMA,
               pltpu.SemaphoreType.DMA,
               pltpu.SemaphoreType.DMA,
               pltpu.SemaphoreType.DMA]

    @functools.partial(
        pl.kernel, mesh=mesh, out_type=out_type, scratch_types=scratch,
        compiler_params=pltpu.CompilerParams(use_tc_tiling_on_sc=False,
                                             needs_layout_passes=False))
    def body(f0, f1, f2, f3, src, dst, mh, zero, o0, o1, o2, o3,
             sidx0, sidx1, didx0, didx1, mrow, rows0, rows1, acc,
             sem0, sem1, ssem0, ssem1):
        c = lax.axis_index("c")
        s = lax.axis_index("s")
        r0 = pl.multiple_of(s * _RPB, 8)
        sidx = [sidx0, sidx1]
        didx = [didx0, didx1]
        rows = [rows0, rows1]
        sems = [sem0, sem1]
        ssems = [ssem0, ssem1]

        def run(tbl, out):
            pltpu.sync_copy(zero.at[pl.ds(r0, _RPB)], acc.at[pl.ds(r0, _RPB)])

            @pl.when(s == _NSUB - 1)
            def _():
                pltpu.sync_copy(zero.at[pl.ds(_NSUB * _RPB, _RTAIL)],
                                acc.at[pl.ds(_NSUB * _RPB, _RTAIL)])

            plsc.subcore_barrier()
            base0 = pl.multiple_of(s * _EPT, 8)
            pltpu.sync_copy(src.at[pl.ds(base0, _CH)], sidx[0])
            handles = [pltpu.async_copy(tbl.at[sidx[0]], rows[0], sems[0]),
                       None]
            scat = [None, None]
            for i in range(_NCHUNK):
                cur = i % 2
                nxt = 1 - cur
                if i + 1 < _NCHUNK:
                    basen = pl.multiple_of(s * _EPT + (i + 1) * _CH, 8)
                    pltpu.sync_copy(src.at[pl.ds(basen, _CH)], sidx[nxt])
                    if scat[nxt] is not None:
                        scat[nxt].wait()
                        scat[nxt] = None
                    handles[nxt] = pltpu.async_copy(tbl.at[sidx[nxt]],
                                                    rows[nxt], sems[nxt])
                base = pl.multiple_of(s * _EPT + i * _CH, 8)
                pltpu.sync_copy(dst.at[pl.ds(base, _CH)], didx[cur])
                handles[cur].wait()
                if masked:
                    mbase = pl.multiple_of(
                        jnp.where(base >= _HALF, base - _HALF, base), 8)
                    pltpu.sync_copy(mh.at[pl.ds(mbase, _CH)], mrow)
                    rc = rows[cur]

                    def scale(e, carry):
                        mvec = plsc.load_gather(
                            mrow, [jnp.full((_LANES,), e, jnp.int32)])
                        for j in range(_DQ // _LANES):
                            sl = pl.ds(j * _LANES, _LANES)
                            rc[e, sl] = rc[e, sl] * mvec
                        return carry

                    lax.fori_loop(0, _CH, scale, 0)
                scat[cur] = pltpu.async_copy(rows[cur], acc.at[didx[cur]],
                                             ssems[cur], add=True)
            for b in range(2):
                if scat[b] is not None:
                    scat[b].wait()
            plsc.subcore_barrier()
            pltpu.sync_copy(acc.at[pl.ds(r0, _RPB)], out.at[pl.ds(r0, _RPB)])

            @pl.when(s == _NSUB - 1)
            def _():
                pltpu.sync_copy(acc.at[pl.ds(_NSUB * _RPB, _RTAIL)],
                                out.at[pl.ds(_NSUB * _RPB, _RTAIL)])

            plsc.subcore_barrier()

        @pl.when(c == 0)
        def _():
            run(f0, o0)
            run(f1, o1)

        @pl.when(c == 1)
        def _():
            run(f2, o2)
            run(f3, o3)

    return body


_segsum_plain = _make_segsum(False)
_segsum_masked = _make_segsum(True)


def _edge_gather(AB, src, dst):
    mesh = plsc.VectorSubcoreMesh(core_axis_name="c", subcore_axis_name="s")
    out_type = [jax.ShapeDtypeStruct((_HALF, 2 * _H), jnp.float32),
                jax.ShapeDtypeStruct((_HALF, 2 * _H), jnp.float32)]
    scratch = [pltpu.VMEM((_CH2,), jnp.int32),
               pltpu.VMEM((_CH2,), jnp.int32),
               pltpu.VMEM((_CH2, 2 * _H), jnp.float32),
               pltpu.VMEM((_CH2, 2 * _H), jnp.float32),
               pltpu.SemaphoreType.DMA,
               pltpu.SemaphoreType.DMA]

    @functools.partial(
        pl.kernel, mesh=mesh, out_type=out_type, scratch_types=scratch,
        compiler_params=pltpu.CompilerParams(use_tc_tiling_on_sc=False,
                                             needs_layout_passes=False))
    def body(AB_h, src_h, dst_h, ABs_o, ABd_o,
             idx0, idx1, rows0, rows1, sem0, sem1):
        c = lax.axis_index("c")
        s = lax.axis_index("s")
        u = s
        idxs = [idx0, idx1]
        rows = [rows0, rows1]
        sems = [sem0, sem1]

        def run(eidx, out):
            base0 = pl.multiple_of(u * _CH2, 8)
            pltpu.sync_copy(eidx.at[pl.ds(base0, _CH2)], idxs[0])
            handles = [pltpu.async_copy(AB_h.at[idxs[0]], rows[0], sems[0]),
                       None]
            for k in range(_NCHUNK2):
                cur = k % 2
                nxt = 1 - cur
                if k + 1 < _NCHUNK2:
                    basen = pl.multiple_of((u + (k + 1) * _NSUB) * _CH2, 8)
                    pltpu.sync_copy(eidx.at[pl.ds(basen, _CH2)], idxs[nxt])
                    handles[nxt] = pltpu.async_copy(AB_h.at[idxs[nxt]],
                                                    rows[nxt], sems[nxt])
                base = pl.multiple_of((u + k * _NSUB) * _CH2, 8)
                handles[cur].wait()
                pltpu.sync_copy(rows[cur], out.at[pl.ds(base, _CH2)])

        @pl.when(c == 0)
        def _():
            run(src_h, ABs_o)

        @pl.when(c == 1)
        def _():
            run(dst_h, ABd_o)

    return body(AB, src, dst)


def _node_mlp_body(a0, a1, a2, a3, W1, We1, AB_o):
    w1 = W1[...]
    we1 = We1[...]
    e = jnp.maximum(
        jnp.dot(a0[...], w1[0 * _DQ:1 * _DQ], preferred_element_type=jnp.float32)
        + jnp.dot(a1[...], w1[1 * _DQ:2 * _DQ], preferred_element_type=jnp.float32)
        + jnp.dot(a2[...], w1[2 * _DQ:3 * _DQ], preferred_element_type=jnp.float32)
        + jnp.dot(a3[...], w1[3 * _DQ:4 * _DQ], preferred_element_type=jnp.float32),
        0.0)
    AB_o[:, :_H] = jnp.dot(e, we1[:_D], preferred_element_type=jnp.float32)
    AB_o[:, _H:] = jnp.dot(e, we1[_D:], preferred_element_type=jnp.float32)


def _node_mlp(aggs, W1, We1):
    n_blk = _N // _BLKN
    qspec = pl.BlockSpec((_BLKN, _DQ), lambda i: (i, 0))
    return pl.pallas_call(
        _node_mlp_body,
        grid=(n_blk,),
        in_specs=[
            qspec, qspec, qspec, qspec,
            pl.BlockSpec((_D, _D), lambda i: (0, 0)),
            pl.BlockSpec((2 * _D, _H), lambda i: (0, 0)),
        ],
        out_specs=pl.BlockSpec((_BLKN, 2 * _H), lambda i: (i, 0)),
        out_shape=jax.ShapeDtypeStruct((_N, 2 * _H), jnp.float32),
    )(*aggs, W1, We1)


def _mask_mlp_body(abs_, abd_, be1, we2, be2, out):
    h1 = jnp.maximum(abs_[:, :_H] + abd_[:, _H:] + be1[...], 0.0)
    v1 = jnp.sum(h1 * we2[...], axis=1, keepdims=True) + be2[0, 0]
    h2 = jnp.maximum(abd_[:, :_H] + abs_[:, _H:] + be1[...], 0.0)
    v2 = jnp.sum(h2 * we2[...], axis=1, keepdims=True) + be2[0, 0]
    out[...] = (jax.nn.sigmoid(v1) + jax.nn.sigmoid(v2)) * 0.5


def _mask_mlp(ABs, ABd, be1, We2, be2):
    n_blk = _HALF // _BLKE
    espec = pl.BlockSpec((_BLKE, 2 * _H), lambda i: (i, 0))
    return pl.pallas_call(
        _mask_mlp_body,
        grid=(n_blk,),
        in_specs=[
            espec, espec,
            pl.BlockSpec((1, _H), lambda i: (0, 0)),
            pl.BlockSpec((1, _H), lambda i: (0, 0)),
            pl.BlockSpec((1, 1), lambda i: (0, 0)),
        ],
        out_specs=pl.BlockSpec((_BLKE, 1), lambda i: (i, 0)),
        out_shape=jax.ShapeDtypeStruct((_HALF, 1), jnp.float32),
    )(ABs, ABd, be1, We2, be2)


def _final_body(a0, a1, a2, a3, W1, W2, out, acc):
    i = pl.program_id(0)
    w1 = W1[...]
    h = jnp.maximum(
        jnp.dot(a0[...], w1[0 * _DQ:1 * _DQ], preferred_element_type=jnp.float32)
        + jnp.dot(a1[...], w1[1 * _DQ:2 * _DQ], preferred_element_type=jnp.float32)
        + jnp.dot(a2[...], w1[2 * _DQ:3 * _DQ], preferred_element_type=jnp.float32)
        + jnp.dot(a3[...], w1[3 * _DQ:4 * _DQ], preferred_element_type=jnp.float32),
        0.0)
    cs = jnp.sum(h, axis=0, keepdims=True)

    @pl.when(i == 0)
    def _():
        acc[...] = cs

    @pl.when(i > 0)
    def _():
        acc[...] = acc[...] + cs

    @pl.when(i == pl.num_programs(0) - 1)
    def _():
        pooled = acc[...] * jnp.float32(1.0 / _N)
        logits = jnp.dot(pooled, W2[...], preferred_element_type=jnp.float32)
        m = jnp.max(logits)
        ex = jnp.exp(logits - m)
        out[...] = ex / jnp.sum(ex)


def _final(aggs, W1, W2):
    n_blk = _N // _BLKN
    qspec = pl.BlockSpec((_BLKN, _DQ), lambda i: (i, 0))
    return pl.pallas_call(
        _final_body,
        grid=(n_blk,),
        in_specs=[
            qspec, qspec, qspec, qspec,
            pl.BlockSpec((_D, _D), lambda i: (0, 0)),
            pl.BlockSpec((_D, _C), lambda i: (0, 0)),
        ],
        out_specs=pl.BlockSpec((1, _C), lambda i: (0, 0)),
        out_shape=jax.ShapeDtypeStruct((1, _C), jnp.float32),
        scratch_shapes=[pltpu.VMEM((1, _D), jnp.float32)],
    )(*aggs, W1, W2)


def kernel(feat, edge_index, W1, W2, We1, be1, We2, be2):
    src = edge_index[0].astype(jnp.int32)
    dst = edge_index[1].astype(jnp.int32)
    fq = [feat[:, k * _DQ:(k + 1) * _DQ] for k in range(4)]
    zero = jnp.zeros((_N, _DQ), jnp.float32)
    dummy_m = jnp.zeros((_HALF,), jnp.float32)

    aggs = _segsum_plain(*fq, src, dst, dummy_m, zero)
    AB = _node_mlp(aggs, W1, We1)
    ABs, ABd = _edge_gather(AB, src, dst)
    m2 = _mask_mlp(ABs, ABd, be1.reshape(1, _H), We2.reshape(1, _H),
                   be2.reshape(1, 1))
    m_h = m2.reshape(-1)
    edge_mask = jnp.concatenate([m_h, m_h])
    a2 = _segsum_masked(*fq, src, dst, m_h, zero)
    probs = _final(a2, W1, W2).reshape(-1)
    return probs, edge_mask
